# Initial kernel scaffold; baseline (speedup 1.0000x reference)
import jax
import jax.numpy as jnp
from jax.experimental import pallas as pl


def kernel(inputs, table, bias):
    raise NotImplementedError("write your pallas kernel here")


# speedup vs baseline: 13.9543x; 13.9543x over previous
import functools

import jax
import jax.numpy as jnp
from jax import lax
from jax.experimental import pallas as pl
from jax.experimental.pallas import tpu as pltpu
from jax.experimental.pallas import tpu_sc as plsc

BATCH = 4096
SEQ = 200
D = 128
NC = 2
NS = 16
NW = NC * NS
SPW = BATCH // NW
C0 = 128
C1 = SEQ - C0
NBUF = 2
NLG = D // 16


def _bow_body(inputs_hbm, table_hbm, bias_hbm, out_hbm,
              idx_slab, rows, out_slab, bias_v, sem0, sem1):
    wid = lax.axis_index("s") * NC + lax.axis_index("c")
    base = wid * SPW
    pltpu.sync_copy(inputs_hbm.at[pl.ds(base, SPW)], idx_slab)
    pltpu.sync_copy(bias_hbm, bias_v)
    sems = (sem0, sem1)

    def issue(i, buf):
        pltpu.async_copy(table_hbm.at[idx_slab.at[i, pl.ds(0, C0)]],
                         rows.at[buf, pl.ds(0, C0)], sems[buf])
        pltpu.async_copy(table_hbm.at[idx_slab.at[i, pl.ds(C0, C1)]],
                         rows.at[buf, pl.ds(C0, C1)], sems[buf])

    def wait(buf):
        pltpu.make_async_copy(table_hbm.at[pl.ds(0, C0)],
                              rows.at[buf, pl.ds(0, C0)], sems[buf]).wait()
        pltpu.make_async_copy(table_hbm.at[pl.ds(0, C1)],
                              rows.at[buf, pl.ds(C0, C1)], sems[buf]).wait()

    issue(0, 0)

    def outer(k, carry):
        i0 = k * NBUF
        for b in range(NBUF):
            i = i0 + b

            @pl.when(i + 1 < SPW)
            def _():
                issue(i + 1, (b + 1) % NBUF)

            wait(b)
            accs = tuple(bias_v[pl.ds(j * 16, 16)] for j in range(NLG))

            def rbody(r, a):
                return tuple(a[j] + rows[b, r, pl.ds(j * 16, 16)]
                             for j in range(NLG))

            accs = lax.fori_loop(0, SEQ, rbody, accs, unroll=4)
            for j in range(NLG):
                out_slab[i, pl.ds(j * 16, 16)] = accs[j]
        return carry

    lax.fori_loop(0, SPW // NBUF, outer, 0)
    pltpu.sync_copy(out_slab, out_hbm.at[pl.ds(base, SPW)])


@jax.jit
def _bow(inputs, table, bias):
    kfn = pl.kernel(
        _bow_body,
        out_type=jax.ShapeDtypeStruct((BATCH, D), jnp.float32),
        mesh=plsc.VectorSubcoreMesh(core_axis_name="c", subcore_axis_name="s"),
        scratch_types=[
            pltpu.VMEM((SPW, SEQ), jnp.int32),
            pltpu.VMEM((NBUF, SEQ, D), jnp.float32),
            pltpu.VMEM((SPW, D), jnp.float32),
            pltpu.VMEM((D,), jnp.float32),
            pltpu.SemaphoreType.DMA,
            pltpu.SemaphoreType.DMA,
        ],
    )
    return kfn(inputs, table, bias)


def kernel(inputs, table, bias):
    return _bow(inputs.astype(jnp.int32), table, bias)
